# baseline (speedup 1.0000x reference)
import functools

import jax
import jax.numpy as jnp
from jax import lax
from jax.experimental import pallas as pl
from jax.experimental.pallas import tpu as pltpu
from jax.experimental.pallas import tpu_sc as plsc

_NC = 2
_NS = 16
_NW = _NC * _NS


def _build(B, F, V, D):
    N = B * F
    CHUNK = 128
    bw = B // _NW
    n_ch = bw // CHUNK
    assert B % (_NW * CHUNK) == 0

    mesh = plsc.VectorSubcoreMesh(core_axis_name="c", subcore_axis_name="s")

    @functools.partial(
        pl.kernel,
        out_type=jax.ShapeDtypeStruct((N, D), jnp.float32),
        mesh=mesh,
        compiler_params=pltpu.CompilerParams(use_tc_tiling_on_sc=False),
        scratch_types=[
            pltpu.VMEM((bw,), jnp.int32),
            pltpu.VMEM((bw, D), jnp.float32),
            pltpu.SemaphoreType.DMA,
        ],
    )
    def k(idx_hbm, *rest):
        tabs = rest[:F]
        out_hbm, idx_v, rows_v, sem = rest[F:]
        wid = lax.axis_index("s") * _NC + lax.axis_index("c")

        for f in range(F):
            base = f * B + wid * bw
            pltpu.sync_copy(idx_hbm.at[pl.ds(base, bw)], idx_v)
            copies = [
                pltpu.async_copy(
                    tabs[f].at[idx_v.at[pl.ds(j * CHUNK, CHUNK)]],
                    rows_v.at[pl.ds(j * CHUNK, CHUNK)],
                    sem,
                )
                for j in range(n_ch)
            ]
            for c in copies:
                c.wait()
            pltpu.sync_copy(rows_v, out_hbm.at[pl.ds(base, bw)])

    return k


def kernel(indices, tables):
    B, F = indices.shape
    _, V, D = tables.shape
    idx_fm = indices.T.reshape(F * B)
    tabs = [tables[f] for f in range(F)]
    out = _build(B, F, V, D)(idx_fm, *tabs)
    return jnp.swapaxes(out.reshape(F, B, D), 0, 1).reshape(B, F * D)


# speedup vs baseline: 1.4174x; 1.4174x over previous
import functools

import jax
import jax.numpy as jnp
from jax import lax
from jax.experimental import pallas as pl
from jax.experimental.pallas import tpu as pltpu
from jax.experimental.pallas import tpu_sc as plsc

_NC = 2
_NS = 16
_NW = _NC * _NS


def _build(B, F, V, D):
    CHUNK = 128
    bw = B // _NW
    n_ch = bw // CHUNK
    assert B % (_NW * CHUNK) == 0

    mesh = plsc.VectorSubcoreMesh(core_axis_name="c", subcore_axis_name="s")

    @functools.partial(
        pl.kernel,
        out_type=jax.ShapeDtypeStruct((B, F, D), jnp.float32),
        mesh=mesh,
        compiler_params=pltpu.CompilerParams(use_tc_tiling_on_sc=False),
        scratch_types=[
            pltpu.VMEM((bw,), jnp.int32),
            pltpu.VMEM((bw,), jnp.int32),
            pltpu.VMEM((bw, D), jnp.float32),
            pltpu.VMEM((bw, D), jnp.float32),
            pltpu.SemaphoreType.DMA,
            pltpu.SemaphoreType.DMA,
        ],
    )
    def k(idx_hbm, tab_hbm, out_hbm, i0, i1, r0, r1, gsem, osem):
        wid = lax.axis_index("s") * _NC + lax.axis_index("c")
        b0 = wid * bw
        idx_bufs = (i0, i1)
        row_bufs = (r0, r1)
        out_pend = [None, None]

        pltpu.sync_copy(idx_hbm.at[pl.ds(b0, bw)], i0)
        for f in range(F):
            p = f & 1
            if out_pend[p] is not None:
                out_pend[p].wait()
                out_pend[p] = None
            gathers = [
                pltpu.async_copy(
                    tab_hbm.at[f].at[idx_bufs[p].at[pl.ds(j * CHUNK, CHUNK)]],
                    row_bufs[p].at[pl.ds(j * CHUNK, CHUNK)],
                    gsem,
                )
                for j in range(n_ch)
            ]
            if f + 1 < F:
                pltpu.sync_copy(
                    idx_hbm.at[pl.ds((f + 1) * B + b0, bw)], idx_bufs[1 - p]
                )
            for c in gathers:
                c.wait()
            out_pend[p] = pltpu.async_copy(
                row_bufs[p], out_hbm.at[pl.ds(b0, bw), f], osem
            )
        for h in out_pend:
            if h is not None:
                h.wait()

    return k


def kernel(indices, tables):
    B, F = indices.shape
    _, V, D = tables.shape
    idx_fm = indices.T.reshape(F * B)
    out = _build(B, F, V, D)(idx_fm, tables)
    return out.reshape(B, F * D)


# speedup vs baseline: 1.6681x; 1.1769x over previous
import functools

import jax
import jax.numpy as jnp
from jax import lax
from jax.experimental import pallas as pl
from jax.experimental.pallas import tpu as pltpu
from jax.experimental.pallas import tpu_sc as plsc

_NC = 2
_NS = 16
_NW = _NC * _NS


def _build(B, F, V, D):
    N = B * F
    CHUNK = 128
    TW = N // _NW
    MEGA = 8
    n_mega = TW // (MEGA * CHUNK)
    assert TW % (MEGA * CHUNK) == 0

    mesh = plsc.VectorSubcoreMesh(core_axis_name="c", subcore_axis_name="s")

    @functools.partial(
        pl.kernel,
        out_type=jax.ShapeDtypeStruct((N, D), jnp.float32),
        mesh=mesh,
        compiler_params=pltpu.CompilerParams(use_tc_tiling_on_sc=False),
        scratch_types=[
            pltpu.VMEM((TW,), jnp.int32),
            pltpu.VMEM((MEGA * CHUNK, D), jnp.float32),
            pltpu.VMEM((MEGA * CHUNK, D), jnp.float32),
            pltpu.SemaphoreType.DMA,
            pltpu.SemaphoreType.DMA,
            pltpu.SemaphoreType.DMA,
        ],
    )
    def k(idx_hbm, tab_hbm, out_hbm, idx_v, r0, r1, gs0, gs1, osem):
        wid = lax.axis_index("s") * _NC + lax.axis_index("c")
        base = wid * TW
        rows = (r0, r1)
        gsems = (gs0, gs1)
        pend_g = [None, None]
        pend_o = [None, None]

        pltpu.sync_copy(idx_hbm.at[pl.ds(base, TW)], idx_v)

        def issue(m):
            pb = m & 1
            pend_g[pb] = [
                pltpu.async_copy(
                    tab_hbm.at[idx_v.at[pl.ds((m * MEGA + j) * CHUNK, CHUNK)]],
                    rows[pb].at[pl.ds(j * CHUNK, CHUNK)],
                    gsems[pb],
                )
                for j in range(MEGA)
            ]

        issue(0)
        for m in range(n_mega):
            pb = m & 1
            if m + 1 < n_mega:
                if pend_o[1 - pb] is not None:
                    pend_o[1 - pb].wait()
                    pend_o[1 - pb] = None
                issue(m + 1)
            for c in pend_g[pb]:
                c.wait()
            pend_o[pb] = pltpu.async_copy(
                rows[pb],
                out_hbm.at[pl.ds(base + m * MEGA * CHUNK, MEGA * CHUNK)],
                osem,
            )
        for h in pend_o:
            if h is not None:
                h.wait()

    return k


def kernel(indices, tables):
    B, F = indices.shape
    _, V, D = tables.shape
    adj = (indices + jnp.arange(F, dtype=indices.dtype)[None, :] * V).reshape(-1)
    tab_flat = tables.reshape(F * V, D)
    out = _build(B, F, V, D)(adj, tab_flat)
    return out.reshape(B, F * D)
